# baseline (speedup 1.0000x reference)
import functools

import jax
import jax.numpy as jnp
from jax import lax
from jax.experimental import pallas as pl
from jax.experimental.pallas import tpu as pltpu
from jax.experimental.pallas import tpu_sc as plsc

N = 16384
C_CAT = 26
C_NUM = 13
V = 100000
D = 16
NC = 2
NS = 16
NW = NC * NS

IDX_P = 4096
TBLK = 2048


def _lin_body(num_ref, w_ref, out_ref):
    out_ref[...] = lax.dot_general(
        w_ref[0], num_ref[...],
        (((0,), (0,)), ((), ())),
        precision=lax.Precision.HIGHEST,
        preferred_element_type=jnp.float32)


def _lin(numT, lw2):
    return pl.pallas_call(
        _lin_body,
        out_shape=jax.ShapeDtypeStruct((NW, N), jnp.float32),
        grid=(2, N // TBLK),
        in_specs=[
            pl.BlockSpec((D, TBLK), lambda t, j: (t, j)),
            pl.BlockSpec((1, D, D), lambda t, j: (t, 0, 0)),
        ],
        out_specs=pl.BlockSpec((D, TBLK), lambda t, j: (t, j)),
    )(numT, lw2)


def _sc_body(embT_u, embT_i, idxT, linT, out2,
             acc_v, ib0, ib1, sem, vsem, asem):
    wid = lax.axis_index("s") * NC + lax.axis_index("c")
    t = wid // D
    d = wid % D

    acp = pltpu.async_copy(linT.at[wid], acc_v, asem)

    NPIECE = N // IDX_P
    NG = C_CAT * NPIECE
    ib = [ib0, ib1]
    isems = [sem, vsem]

    def istart(g):
        c, p = divmod(g, NPIECE)
        return pltpu.async_copy(
            idxT.at[t, c, pl.ds(p * IDX_P, IDX_P)], ib[g % 2], isems[g % 2])

    def main_scope(vec_v):
        istart(0)
        for c in range(C_CAT):
            row = c * D + d

            @pl.when(t == 0)
            def _():
                pltpu.sync_copy(embT_u.at[row], vec_v)

            @pl.when(t > 0)
            def _():
                pltpu.sync_copy(embT_i.at[row], vec_v)

            if c == 0:
                acp.wait()

            for h in range(NPIECE):
                g = c * NPIECE + h
                pltpu.make_async_copy(
                    idxT.at[t, 0, pl.ds(0, IDX_P)], ib[g % 2],
                    isems[g % 2]).wait()
                if g + 1 < NG:
                    istart(g + 1)
                idx_v = ib[g % 2]
                base = h * IDX_P

                def gbody(j, _):
                    idxv = idx_v[pl.ds(j * D, D)]
                    g2 = plsc.load_gather(vec_v, [idxv])
                    a = base + j * D
                    acc_v[pl.ds(a, D)] = acc_v[pl.ds(a, D)] + g2
                    return 0

                lax.fori_loop(0, IDX_P // D, gbody, 0, unroll=8)

    pl.run_scoped(main_scope, pltpu.VMEM((V,), jnp.float32))

    pltpu.sync_copy(acc_v, out2.at[wid])


@jax.jit
def _run(embT_u, embT_i, idxT, linT):
    mesh = plsc.VectorSubcoreMesh(core_axis_name="c", subcore_axis_name="s")
    return pl.kernel(
        _sc_body,
        out_type=jax.ShapeDtypeStruct((NW, N), jnp.float32),
        mesh=mesh,
        scratch_types=[
            pltpu.VMEM((N,), jnp.float32),
            pltpu.VMEM((IDX_P,), jnp.int32),
            pltpu.VMEM((IDX_P,), jnp.int32),
            pltpu.SemaphoreType.DMA,
            pltpu.SemaphoreType.DMA,
            pltpu.SemaphoreType.DMA,
        ],
        compiler_params=pltpu.CompilerParams(needs_layout_passes=False),
    )(embT_u, embT_i, idxT, linT)


def kernel(cat_user, num_user, cat_item, num_item,
           emb_user, lin_w_user, lin_b_user,
           emb_item, lin_w_item, lin_b_item):
    embT_u = emb_user.swapaxes(1, 2).reshape(C_CAT * D, V)
    embT_i = emb_item.swapaxes(1, 2).reshape(C_CAT * D, V)
    idxT = jnp.stack([cat_user.astype(jnp.int32).T,
                      cat_item.astype(jnp.int32).T])
    ones = jnp.ones((1, N), jnp.float32)
    zer = jnp.zeros((D - C_NUM - 1, N), jnp.float32)
    numT = jnp.concatenate(
        [num_user.T, ones, zer, num_item.T, ones, zer], axis=0)
    zw = jnp.zeros((D - C_NUM - 1, D), jnp.float32)
    lw_u = jnp.concatenate([lin_w_user, lin_b_user.sum(0)[None], zw], axis=0)
    lw_i = jnp.concatenate([lin_w_item, lin_b_item.sum(0)[None], zw], axis=0)
    lw2 = jnp.stack([lw_u, lw_i])
    linT = _lin(numT, lw2)
    out2 = _run(embT_u, embT_i, idxT, linT)
    return out2.reshape(2, D, N).swapaxes(1, 2)
